# Initial kernel scaffold; baseline (speedup 1.0000x reference)
import jax
import jax.numpy as jnp
from jax.experimental import pallas as pl


def kernel(energy, forces, atomic_numbers, idx_s, idx_t, batch, W1, b1, W2, b2, We1, be1, We2, be2, per_atom_scales, per_atom_shifts, pairwise_scales):
    raise NotImplementedError("write your pallas kernel here")


# speedup vs baseline: 15.2892x; 15.2892x over previous
import functools

import jax
import jax.numpy as jnp
from jax import lax
from jax.experimental import pallas as pl
from jax.experimental.pallas import tpu as pltpu
from jax.experimental.pallas import tpu_sc as plsc

N = 100000
E = 1600000
D = 128
DE = 16
Z = 101
B = 16

ZP = 128
PWP = 10208

NW = 32
EC = 2000

BN = 2000
BE = 8000


def _sc_body(an_hbm, is_hbm, it_hbm, pw_hbm, sc_hbm, m_hbm,
             an_v, pw_v, sc_v, is_v, it_v, m_v):
    wid = lax.axis_index("s") * 2 + lax.axis_index("c")
    pltpu.sync_copy(an_hbm, an_v)
    pltpu.sync_copy(pw_hbm, pw_v)
    pltpu.sync_copy(sc_hbm, sc_v)
    per_w = E // NW
    base = wid * per_w

    def chunk(ci, carry):
        g = base + ci * EC
        pltpu.sync_copy(is_hbm.at[pl.ds(g, EC)], is_v)
        pltpu.sync_copy(it_hbm.at[pl.ds(g, EC)], it_v)

        def vec(j, carry2):
            o = j * 16
            i_s = is_v[pl.ds(o, 16)]
            i_t = it_v[pl.ds(o, 16)]
            a_s = plsc.load_gather(an_v, [i_s])
            a_t = plsc.load_gather(an_v, [i_t])
            pwv = plsc.load_gather(pw_v, [a_s * Z + a_t])
            scv = plsc.load_gather(sc_v, [a_t])
            m_v[pl.ds(o, 16)] = pwv * scv
            return carry2

        lax.fori_loop(0, EC // 16, vec, 0)
        pltpu.sync_copy(m_v, m_hbm.at[pl.ds(g, EC)])
        return carry

    lax.fori_loop(0, per_w // EC, chunk, 0)


_sc_multipliers = functools.partial(
    pl.kernel,
    out_type=jax.ShapeDtypeStruct((E,), jnp.float32),
    mesh=plsc.VectorSubcoreMesh(core_axis_name="c", subcore_axis_name="s"),
    compiler_params=pltpu.CompilerParams(needs_layout_passes=False),
    scratch_types=[
        pltpu.VMEM((N,), jnp.int32),
        pltpu.VMEM((PWP,), jnp.float32),
        pltpu.VMEM((ZP,), jnp.float32),
        pltpu.VMEM((EC,), jnp.int32),
        pltpu.VMEM((EC,), jnp.int32),
        pltpu.VMEM((EC,), jnp.float32),
    ],
)(_sc_body)


def _node_body(e_ref, an_ref, b_ref, w1_ref, b1_ref, w2_ref, b2_ref,
               ss_ref, out_ref, cnt_ref):
    h = jax.nn.silu(
        jnp.dot(e_ref[...], w1_ref[...], preferred_element_type=jnp.float32)
        + b1_ref[...])
    pae = jnp.dot(h, w2_ref[...], preferred_element_type=jnp.float32) + b2_ref[...]
    oh = (an_ref[...] == lax.broadcasted_iota(jnp.int32, (BN, ZP), 1)
          ).astype(jnp.float32)
    ssh = jnp.dot(oh, ss_ref[...], preferred_element_type=jnp.float32)
    pa = pae * ssh[:, 0:1] + ssh[:, 1:2]
    boh = (b_ref[...] == lax.broadcasted_iota(jnp.int32, (BN, B), 1)
           ).astype(jnp.float32)

    @pl.when(pl.program_id(0) == 0)
    def _():
        out_ref[...] = jnp.zeros_like(out_ref)
        cnt_ref[...] = jnp.zeros_like(cnt_ref)

    out_ref[...] += jnp.sum(pa * boh, axis=0, keepdims=True)
    cnt_ref[...] += jnp.sum(boh, axis=0, keepdims=True)


def _edge_body(f_ref, m_ref, it_ref, st_ref, en_ref, we1_ref, be1_ref,
               we2_ref, be2_ref, out_ref):
    h = jax.nn.silu(
        jnp.dot(f_ref[...], we1_ref[...], preferred_element_type=jnp.float32)
        + be1_ref[...])
    pe = jnp.dot(h, we2_ref[...], preferred_element_type=jnp.float32) + be2_ref[...]
    v = pe * m_ref[...]
    it = it_ref[...]
    oh = ((it >= st_ref[...]) & (it < en_ref[...])).astype(jnp.float32)

    @pl.when(pl.program_id(0) == 0)
    def _():
        out_ref[...] = jnp.zeros_like(out_ref)

    out_ref[...] += jnp.sum(v * oh, axis=0, keepdims=True)


def kernel(energy, forces, atomic_numbers, idx_s, idx_t, batch,
           W1, b1, W2, b2, We1, be1, We2, be2,
           per_atom_scales, per_atom_shifts, pairwise_scales):
    an = atomic_numbers.astype(jnp.int32)
    pw_pad = jnp.pad(pairwise_scales[:, 0], (0, PWP - Z * Z))
    sc_pad = jnp.pad(per_atom_scales[:, 0], (0, ZP - Z))
    ss_tab = jnp.stack(
        [jnp.pad(per_atom_scales[:, 0], (0, ZP - Z)),
         jnp.pad(per_atom_shifts[:, 0], (0, ZP - Z))], axis=1)

    m = _sc_multipliers(an, idx_s, idx_t, pw_pad, sc_pad)

    node16, cnt16 = pl.pallas_call(
        _node_body,
        grid=(N // BN,),
        in_specs=[
            pl.BlockSpec((BN, D), lambda i: (i, 0)),
            pl.BlockSpec((BN, 1), lambda i: (i, 0)),
            pl.BlockSpec((BN, 1), lambda i: (i, 0)),
            pl.BlockSpec((D, D), lambda i: (0, 0)),
            pl.BlockSpec((1, D), lambda i: (0, 0)),
            pl.BlockSpec((D, 1), lambda i: (0, 0)),
            pl.BlockSpec((1, 1), lambda i: (0, 0)),
            pl.BlockSpec((ZP, 2), lambda i: (0, 0)),
        ],
        out_specs=[
            pl.BlockSpec((1, B), lambda i: (0, 0)),
            pl.BlockSpec((1, B), lambda i: (0, 0)),
        ],
        out_shape=[
            jax.ShapeDtypeStruct((1, B), jnp.float32),
            jax.ShapeDtypeStruct((1, B), jnp.float32),
        ],
    )(energy, an[:, None], batch[:, None], W1, b1[None, :], W2,
      b2[None, :], ss_tab)

    ends = jnp.cumsum(cnt16[0].astype(jnp.int32))
    starts = jnp.concatenate([jnp.zeros((1,), jnp.int32), ends[:-1]])

    edge16 = pl.pallas_call(
        _edge_body,
        grid=(E // BE,),
        in_specs=[
            pl.BlockSpec((BE, DE), lambda i: (i, 0)),
            pl.BlockSpec((BE, 1), lambda i: (i, 0)),
            pl.BlockSpec((BE, 1), lambda i: (i, 0)),
            pl.BlockSpec((1, B), lambda i: (0, 0)),
            pl.BlockSpec((1, B), lambda i: (0, 0)),
            pl.BlockSpec((DE, DE), lambda i: (0, 0)),
            pl.BlockSpec((1, DE), lambda i: (0, 0)),
            pl.BlockSpec((DE, 1), lambda i: (0, 0)),
            pl.BlockSpec((1, 1), lambda i: (0, 0)),
        ],
        out_specs=pl.BlockSpec((1, B), lambda i: (0, 0)),
        out_shape=jax.ShapeDtypeStruct((1, B), jnp.float32),
    )(forces, m[:, None], idx_t[:, None], starts[None, :], ends[None, :],
      We1, be1[None, :], We2, be2[None, :])

    return (node16 + edge16)[0]


# speedup vs baseline: 48.4500x; 3.1689x over previous
import functools

import jax
import jax.numpy as jnp
from jax import lax
from jax.experimental import pallas as pl
from jax.experimental.pallas import tpu as pltpu
from jax.experimental.pallas import tpu_sc as plsc

N = 100000
E = 1600000
D = 128
DE = 16
Z = 101
B = 16

ZP = 128
PWP = 10208

NW = 32
EC = 2000

BN = 2000
BE = 16000


def _sc_body(an_hbm, is_hbm, it_hbm, pw_hbm, sc_hbm, m_hbm,
             an_v, pw_v, sc_v, is_v, it_v, m_v):
    wid = lax.axis_index("s") * 2 + lax.axis_index("c")
    pltpu.sync_copy(an_hbm, an_v)
    pltpu.sync_copy(pw_hbm, pw_v)
    pltpu.sync_copy(sc_hbm, sc_v)
    per_w = E // NW
    base = wid * per_w

    def chunk(ci, carry):
        g = base + ci * EC
        pltpu.sync_copy(is_hbm.at[pl.ds(g, EC)], is_v)
        pltpu.sync_copy(it_hbm.at[pl.ds(g, EC)], it_v)

        def vec(j, carry2):
            o = j * 16
            i_s = is_v[pl.ds(o, 16)]
            i_t = it_v[pl.ds(o, 16)]
            a_s = plsc.load_gather(an_v, [i_s])
            a_t = plsc.load_gather(an_v, [i_t])
            pwv = plsc.load_gather(pw_v, [a_s * Z + a_t])
            scv = plsc.load_gather(sc_v, [a_t])
            m_v[pl.ds(o, 16)] = pwv * scv
            return carry2

        lax.fori_loop(0, EC // 16, vec, 0)
        pltpu.sync_copy(m_v, m_hbm.at[pl.ds(g, EC)])
        return carry

    lax.fori_loop(0, per_w // EC, chunk, 0)


_sc_multipliers = functools.partial(
    pl.kernel,
    out_type=jax.ShapeDtypeStruct((E,), jnp.float32),
    mesh=plsc.VectorSubcoreMesh(core_axis_name="c", subcore_axis_name="s"),
    compiler_params=pltpu.CompilerParams(needs_layout_passes=False),
    scratch_types=[
        pltpu.VMEM((N,), jnp.int32),
        pltpu.VMEM((PWP,), jnp.float32),
        pltpu.VMEM((ZP,), jnp.float32),
        pltpu.VMEM((EC,), jnp.int32),
        pltpu.VMEM((EC,), jnp.int32),
        pltpu.VMEM((EC,), jnp.float32),
    ],
)(_sc_body)


def _node_body(e_ref, an_ref, b_ref, w1t_ref, b1c_ref, w2t_ref, b2_ref,
               ss_ref, out_ref, cnt_ref):
    ht = jax.nn.silu(
        lax.dot_general(w1t_ref[...], e_ref[...], (((1,), (1,)), ((), ())),
                        preferred_element_type=jnp.float32)
        + b1c_ref[...])
    pae = jnp.dot(w2t_ref[...], ht,
                  preferred_element_type=jnp.float32) + b2_ref[...]
    an_row = an_ref[0]
    b_row = b_ref[0]
    oh = (an_row == lax.broadcasted_iota(jnp.int32, (ZP, BN), 0)
          ).astype(jnp.float32)
    ssh = jnp.dot(ss_ref[...], oh, preferred_element_type=jnp.float32)
    pa = pae * ssh[0:1, :] + ssh[1:2, :]
    boh = (b_row == lax.broadcasted_iota(jnp.int32, (B, BN), 0)
           ).astype(jnp.float32)

    @pl.when(pl.program_id(0) == 0)
    def _():
        out_ref[...] = jnp.zeros_like(out_ref)
        cnt_ref[...] = jnp.zeros_like(cnt_ref)

    out_ref[...] += jnp.sum(pa * boh, axis=1, keepdims=True)
    cnt_ref[...] += jnp.sum(boh, axis=1, keepdims=True)


def _edge_body(f_ref, m_ref, it_ref, st_ref, we1t_ref, be1c_ref,
               we2t_ref, be2_ref, out_ref):
    ht = jax.nn.silu(
        lax.dot_general(we1t_ref[...], f_ref[...], (((1,), (1,)), ((), ())),
                        preferred_element_type=jnp.float32)
        + be1c_ref[...])
    pe = jnp.dot(we2t_ref[...], ht,
                 preferred_element_type=jnp.float32) + be2_ref[...]
    v = pe * m_ref[0]
    ge = (it_ref[0] >= st_ref[...]).astype(jnp.float32)

    @pl.when(pl.program_id(0) == 0)
    def _():
        out_ref[...] = jnp.zeros_like(out_ref)

    out_ref[...] += jnp.sum(v * ge, axis=1, keepdims=True)


def kernel(energy, forces, atomic_numbers, idx_s, idx_t, batch,
           W1, b1, W2, b2, We1, be1, We2, be2,
           per_atom_scales, per_atom_shifts, pairwise_scales):
    an = atomic_numbers.astype(jnp.int32)
    pw_pad = jnp.pad(pairwise_scales[:, 0], (0, PWP - Z * Z))
    sc_pad = jnp.pad(per_atom_scales[:, 0], (0, ZP - Z))
    ss_tab = jnp.stack(
        [jnp.pad(per_atom_scales[:, 0], (0, ZP - Z)),
         jnp.pad(per_atom_shifts[:, 0], (0, ZP - Z))], axis=0)

    m = _sc_multipliers(an, idx_s, idx_t, pw_pad, sc_pad)

    node16, cnt16 = pl.pallas_call(
        _node_body,
        grid=(N // BN,),
        in_specs=[
            pl.BlockSpec((BN, D), lambda i: (i, 0)),
            pl.BlockSpec((1, 1, BN), lambda i: (i, 0, 0)),
            pl.BlockSpec((1, 1, BN), lambda i: (i, 0, 0)),
            pl.BlockSpec((D, D), lambda i: (0, 0)),
            pl.BlockSpec((D, 1), lambda i: (0, 0)),
            pl.BlockSpec((1, D), lambda i: (0, 0)),
            pl.BlockSpec((1, 1), lambda i: (0, 0)),
            pl.BlockSpec((2, ZP), lambda i: (0, 0)),
        ],
        out_specs=[
            pl.BlockSpec((B, 1), lambda i: (0, 0)),
            pl.BlockSpec((B, 1), lambda i: (0, 0)),
        ],
        out_shape=[
            jax.ShapeDtypeStruct((B, 1), jnp.float32),
            jax.ShapeDtypeStruct((B, 1), jnp.float32),
        ],
    )(energy, an.reshape(N // BN, 1, BN), batch.reshape(N // BN, 1, BN),
      W1.T, b1[:, None], W2.T, b2[None, :], ss_tab)

    ends = jnp.cumsum(cnt16[:, 0].astype(jnp.int32))
    starts = jnp.concatenate([jnp.zeros((1,), jnp.int32), ends[:-1]])

    s16 = pl.pallas_call(
        _edge_body,
        grid=(E // BE,),
        in_specs=[
            pl.BlockSpec((BE, DE), lambda i: (i, 0)),
            pl.BlockSpec((1, 1, BE), lambda i: (i, 0, 0)),
            pl.BlockSpec((1, 1, BE), lambda i: (i, 0, 0)),
            pl.BlockSpec((B, 1), lambda i: (0, 0)),
            pl.BlockSpec((DE, DE), lambda i: (0, 0)),
            pl.BlockSpec((DE, 1), lambda i: (0, 0)),
            pl.BlockSpec((1, DE), lambda i: (0, 0)),
            pl.BlockSpec((1, 1), lambda i: (0, 0)),
        ],
        out_specs=pl.BlockSpec((B, 1), lambda i: (0, 0)),
        out_shape=jax.ShapeDtypeStruct((B, 1), jnp.float32),
    )(forces, m.reshape(E // BE, 1, BE), idx_t.reshape(E // BE, 1, BE),
      starts[:, None], We1.T, be1[:, None], We2.T, be2[None, :])

    s = s16[:, 0]
    edge16 = s - jnp.concatenate([s[1:], jnp.zeros((1,), jnp.float32)])
    return node16[:, 0] + edge16
